# Initial kernel scaffold; baseline (speedup 1.0000x reference)
import jax
import jax.numpy as jnp
from jax.experimental import pallas as pl


def kernel(feature, edge_tc, edge_sc, W, b, edge_index):
    raise NotImplementedError("write your pallas kernel here")


# speedup vs baseline: 5.2958x; 5.2958x over previous
import functools

import jax
import jax.numpy as jnp
from jax import lax
from jax.experimental import pallas as pl
from jax.experimental.pallas import tpu as pltpu
from jax.experimental.pallas import tpu_sc as plsc

N = 50000
E = 1600000
F = 16
OUT = 128
NC, NS = 2, 16
NW = NC * NS
EW = E // NW
B = 2000
NB = EW // B
SB = 80
K = B // SB
NP = 50048
ZR = NP // NS


def _sc_segment_sum(tc2, sc1, dst3):
    mesh = plsc.VectorSubcoreMesh(core_axis_name="c", subcore_axis_name="s")

    @functools.partial(
        pl.kernel,
        out_type=jax.ShapeDtypeStruct((NC, NP, F), jnp.float32),
        mesh=mesh,
        compiler_params=pltpu.CompilerParams(use_tc_tiling_on_sc=False),
        scratch_types=[
            pltpu.VMEM((B, F), jnp.float32),
            pltpu.VMEM((B,), jnp.float32),
            pltpu.VMEM((K, SB), jnp.int32),
            pltpu.VMEM_SHARED((NP, F), jnp.float32),
        ],
    )
    def k(tc_hbm, sc_hbm, dst_hbm, out_hbm, tc_buf, sc_buf, dst_buf, acc):
        c = lax.axis_index("c")
        s = lax.axis_index("s")
        wid = s * NC + c

        @pl.loop(0, B)
        def _z(i):
            tc_buf[i, :] = jnp.zeros((F,), jnp.float32)

        pltpu.sync_copy(tc_buf, acc.at[pl.ds(s * ZR, B), :])
        pltpu.sync_copy(tc_buf.at[pl.ds(0, ZR - B), :],
                        acc.at[pl.ds(s * ZR + B, ZR - B), :])
        plsc.subcore_barrier()

        @pl.loop(0, NB)
        def _blk(blk):
            base = wid * EW + blk * B
            pltpu.sync_copy(tc_hbm.at[pl.ds(base, B), :], tc_buf)
            pltpu.sync_copy(sc_hbm.at[pl.ds(base, B)], sc_buf)
            pltpu.sync_copy(dst_hbm.at[wid * NB + blk], dst_buf)

            @plsc.parallel_loop(0, B // 16, 1, unroll=2)
            def _mul(gi):
                g = sc_buf[pl.ds(gi * 16, 16)]
                for r in range(16):
                    tc_buf[gi * 16 + r, :] = tc_buf[gi * 16 + r, :] * g[r]

            for j in range(K):
                pltpu.sync_copy(tc_buf.at[pl.ds(j * SB, SB), :],
                                acc.at[dst_buf.at[j]], add=True)

        plsc.subcore_barrier()

        pltpu.sync_copy(acc.at[pl.ds(s * ZR, B), :], tc_buf)
        pltpu.sync_copy(tc_buf, out_hbm.at[c, pl.ds(s * ZR, B), :])
        pltpu.sync_copy(acc.at[pl.ds(s * ZR + B, ZR - B), :],
                        tc_buf.at[pl.ds(0, ZR - B), :])
        pltpu.sync_copy(tc_buf.at[pl.ds(0, ZR - B), :],
                        out_hbm.at[c, pl.ds(s * ZR + B, ZR - B), :])

    return k(tc2, sc1, dst3)


def _tc_linear(parts, W, b2):
    R = 400
    G = N // R

    def mm(p_ref, w_ref, b_ref, o_ref):
        a = p_ref[0] + p_ref[1]
        acc = lax.dot_general(a, w_ref[...], (((1,), (1,)), ((), ())),
                              preferred_element_type=jnp.float32)
        o_ref[...] = acc + b_ref[...]

    return pl.pallas_call(
        mm,
        grid=(G,),
        in_specs=[
            pl.BlockSpec((NC, R, F), lambda i: (0, i, 0)),
            pl.BlockSpec((OUT, F), lambda i: (0, 0)),
            pl.BlockSpec((1, OUT), lambda i: (0, 0)),
        ],
        out_specs=pl.BlockSpec((R, OUT), lambda i: (i, 0)),
        out_shape=jax.ShapeDtypeStruct((N, OUT), jnp.float32),
    )(parts, W, b2)


def kernel(feature, edge_tc, edge_sc, W, b, edge_index):
    del feature
    sc1 = edge_sc.reshape(E)
    dst3 = edge_index[1].reshape(NW * NB, K, SB)
    parts = _sc_segment_sum(edge_tc, sc1, dst3)
    return _tc_linear(parts, W, b.reshape(1, OUT))
